# Initial kernel scaffold; baseline (speedup 1.0000x reference)
import functools

import jax
import jax.numpy as jnp
from jax import lax
from jax.experimental import pallas as pl
from jax.experimental.pallas import tpu as pltpu
from jax.experimental.pallas import tpu_sc as plsc

N_NODES = 10000
E = 320000
IN_DIM, HID, LAT = 128, 64, 32
LEAKY_ALPHA = 0.2

NC, NS, L = 2, 16, 16
NW = NC * NS
N_PAD = 10240
UE = 2 * E
C = 128
EPW = 20480
UE_PAD = EPW * NW
NCH = EPW // C
RPT = N_PAD // NS

W1T = 80
W2T = 48

_mesh = plsc.VectorSubcoreMesh(core_axis_name="c", subcore_axis_name="s",
                               num_cores=NC, num_subcores=NS)

BR = 1024
GRID = N_PAD // BR


def _f32(shape):
    return jax.ShapeDtypeStruct(shape, jnp.float32)


@functools.partial(
    pl.kernel, mesh=_mesh,
    out_type=_f32((NC, N_PAD)),
    scratch_types=[
        pltpu.VMEM_SHARED((N_PAD,), jnp.float32),
        pltpu.VMEM((C,), jnp.int32),
        pltpu.VMEM((C,), jnp.float32),
        pltpu.VMEM((RPT,), jnp.float32),
    ],
)
def _deg_kernel(usrc_hbm, degp_hbm, sh_deg, sidx, ones_v, zbuf):
    c = lax.axis_index("c")
    s = lax.axis_index("s")
    wid = s * NC + c

    @pl.loop(0, RPT // L)
    def _(i):
        zbuf[pl.ds(i * L, L)] = jnp.zeros((L,), jnp.float32)

    @pl.loop(0, C // L)
    def _(i):
        ones_v[pl.ds(i * L, L)] = jnp.ones((L,), jnp.float32)

    pltpu.sync_copy(zbuf, sh_deg.at[pl.ds(s * RPT, RPT)])
    plsc.subcore_barrier()

    @pl.loop(0, NCH)
    def _(i):
        base = wid * EPW + i * C
        pltpu.sync_copy(usrc_hbm.at[pl.ds(base, C)], sidx)
        pltpu.sync_copy(ones_v, sh_deg.at[sidx], add=True)

    plsc.subcore_barrier()
    pltpu.sync_copy(sh_deg.at[pl.ds(s * RPT, RPT)],
                    degp_hbm.at[c, pl.ds(s * RPT, RPT)])


def _make_edge_pass(WT):
    @functools.partial(
        pl.kernel, mesh=_mesh,
        out_type=_f32((NC, N_PAD, WT)),
        scratch_types=[
            pltpu.VMEM_SHARED((N_PAD, WT), jnp.float32),
            pltpu.VMEM((N_PAD,), jnp.float32),
            pltpu.VMEM((N_PAD,), jnp.float32),
            pltpu.VMEM((N_PAD,), jnp.float32),
            pltpu.VMEM((C,), jnp.int32),
            pltpu.VMEM((C,), jnp.int32),
            pltpu.VMEM((C, WT), jnp.float32),
            pltpu.VMEM((L,), jnp.float32),
        ],
    )
    def _edge_kernel(usrc_hbm, udst_hbm, degp_hbm, es_hbm, tab_hbm, agg_hbm,
                     sh_agg, es_t, inv_t, tmp_t, sidx, didx, rows, exbuf):
        c = lax.axis_index("c")
        s = lax.axis_index("s")
        wid = s * NC + c

        pltpu.sync_copy(es_hbm, es_t)
        pltpu.sync_copy(degp_hbm.at[0], inv_t)
        pltpu.sync_copy(degp_hbm.at[1], tmp_t)

        @pl.loop(0, N_PAD // L)
        def _(i):
            dsl = pl.ds(i * L, L)
            dv = inv_t[dsl] + tmp_t[dsl]
            inv_t[dsl] = 1.0 / jnp.maximum(dv, 1.0)

        @pl.loop(0, C)
        def _(e):
            for k in range(WT // L):
                rows[e, pl.ds(k * L, L)] = jnp.zeros((L,), jnp.float32)

        for b in range(RPT // C):
            pltpu.sync_copy(rows, sh_agg.at[pl.ds(s * RPT + b * C, C)])
        plsc.subcore_barrier()

        iota = lax.iota(jnp.int32, L)
        col_ed = jnp.full((L,), WT // 2 + 1, jnp.int32)

        @pl.loop(0, NCH)
        def _(i):
            base = wid * EPW + i * C
            pltpu.sync_copy(usrc_hbm.at[pl.ds(base, C)], sidx)
            pltpu.sync_copy(udst_hbm.at[pl.ds(base, C)], didx)
            pltpu.sync_copy(tab_hbm.at[didx], rows)

            @pl.loop(0, C // L)
            def _(j):
                s16 = sidx[pl.ds(j * L, L)]
                r16 = iota + j * L
                es16 = plsc.load_gather(es_t, [s16])
                iv16 = plsc.load_gather(inv_t, [s16])
                ed16 = plsc.load_gather(rows, [r16, col_ed])
                t = es16 + ed16
                t = jnp.maximum(t, LEAKY_ALPHA * t)
                exbuf[...] = jnp.exp(iv16 * t)
                for e in range(L):
                    exs = exbuf[e]
                    row = j * L + e
                    for k in range(WT // L):
                        dsl = pl.ds(k * L, L)
                        rows[row, dsl] = rows[row, dsl] * exs

            pltpu.sync_copy(rows, sh_agg.at[sidx], add=True)

        plsc.subcore_barrier()
        pltpu.sync_copy(sh_agg.at[pl.ds(s * RPT, RPT)],
                        agg_hbm.at[c, pl.ds(s * RPT, RPT)])

    return _edge_kernel


_edge_pass_1 = _make_edge_pass(W1T)
_edge_pass_2 = _make_edge_pass(W2T)


@functools.partial(
    pl.kernel, mesh=_mesh,
    out_type=_f32((UE_PAD,)),
    scratch_types=[
        pltpu.VMEM((C,), jnp.int32),
        pltpu.VMEM((C,), jnp.int32),
        pltpu.VMEM((C, LAT), jnp.float32),
        pltpu.VMEM((C, LAT), jnp.float32),
        pltpu.VMEM((C,), jnp.float32),
    ],
)
def _decoder_kernel(usrc_hbm, udst_hbm, z_hbm, out_hbm, sidx, didx, zs, zd, scb):
    c = lax.axis_index("c")
    s = lax.axis_index("s")
    wid = s * NC + c
    iota = lax.iota(jnp.int32, L)

    @pl.loop(0, NCH)
    def _(i):
        base = wid * EPW + i * C
        pltpu.sync_copy(usrc_hbm.at[pl.ds(base, C)], sidx)
        pltpu.sync_copy(udst_hbm.at[pl.ds(base, C)], didx)
        pltpu.sync_copy(z_hbm.at[sidx], zs)
        pltpu.sync_copy(z_hbm.at[didx], zd)

        @pl.loop(0, C // L)
        def _(j):
            r16 = iota + j * L
            acc = jnp.zeros((L,), jnp.float32)
            for k in range(LAT):
                ck = jnp.full((L,), k, jnp.int32)
                acc = acc + plsc.load_gather(zs, [r16, ck]) * \
                    plsc.load_gather(zd, [r16, ck])
            scb[pl.ds(j * L, L)] = 1.0 / (1.0 + jnp.exp(-acc))

        pltpu.sync_copy(scb, out_hbm.at[pl.ds(base, C)])


def _dot(a, b):
    return lax.dot_general(a, b, (((1,), (0,)), ((), ())),
                           precision=lax.Precision.HIGHEST,
                           preferred_element_type=jnp.float32)


def _tc_a_body(x_ref, w_ref, r_ref, as_ref, ad_ref, tab_ref, res_ref, es_ref):
    x = x_ref[...]
    h = _dot(x, w_ref[...])
    res_ref[...] = _dot(x, r_ref[...])
    es_ref[...] = jnp.sum(h * as_ref[...], axis=1, keepdims=True)
    ed = jnp.sum(h * ad_ref[...], axis=1, keepdims=True)
    ones = jnp.ones((BR, 1), jnp.float32)
    pad = jnp.zeros((BR, W1T - HID - 2), jnp.float32)
    tab_ref[...] = jnp.concatenate([h, ones, ed, pad], axis=1)


def _tc_a(x_pad, w1, r1, as1, ad1):
    return pl.pallas_call(
        _tc_a_body,
        grid=(GRID,),
        in_specs=[
            pl.BlockSpec((BR, IN_DIM), lambda i: (i, 0)),
            pl.BlockSpec((IN_DIM, HID), lambda i: (0, 0)),
            pl.BlockSpec((IN_DIM, HID), lambda i: (0, 0)),
            pl.BlockSpec((1, HID), lambda i: (0, 0)),
            pl.BlockSpec((1, HID), lambda i: (0, 0)),
        ],
        out_specs=[
            pl.BlockSpec((BR, W1T), lambda i: (i, 0)),
            pl.BlockSpec((BR, HID), lambda i: (i, 0)),
            pl.BlockSpec((BR, 1), lambda i: (i, 0)),
        ],
        out_shape=[_f32((N_PAD, W1T)), _f32((N_PAD, HID)), _f32((N_PAD, 1))],
    )(x_pad, w1, r1, as1, ad1)


def _post_layer(agg, res, g, b):
    dim = res.shape[1]
    num = agg[:, :dim]
    den = agg[:, dim:dim + 1]
    h = num / (den + 1e-16) + res
    mu = jnp.mean(h, axis=1, keepdims=True)
    d = h - mu
    var = jnp.mean(d * d, axis=1, keepdims=True)
    hn = d * lax.rsqrt(var + 1e-5) * g + b
    return jnp.where(hn > 0, hn, jnp.exp(hn) - 1.0)


def _tc_b_body(agg_ref, res_ref, g_ref, b_ref, w_ref, r_ref, as_ref, ad_ref,
               tab_ref, res2_ref, es_ref):
    agg = agg_ref[0] + agg_ref[1]
    x2 = _post_layer(agg, res_ref[...], g_ref[...], b_ref[...])
    h2 = _dot(x2, w_ref[...])
    res2_ref[...] = _dot(x2, r_ref[...])
    es_ref[...] = jnp.sum(h2 * as_ref[...], axis=1, keepdims=True)
    ed = jnp.sum(h2 * ad_ref[...], axis=1, keepdims=True)
    ones = jnp.ones((BR, 1), jnp.float32)
    pad = jnp.zeros((BR, W2T - LAT - 2), jnp.float32)
    tab_ref[...] = jnp.concatenate([h2, ones, ed, pad], axis=1)


def _tc_b(agg1p, res1, g1, b1, w2, r2, as2, ad2):
    return pl.pallas_call(
        _tc_b_body,
        grid=(GRID,),
        in_specs=[
            pl.BlockSpec((NC, BR, W1T), lambda i: (0, i, 0)),
            pl.BlockSpec((BR, HID), lambda i: (i, 0)),
            pl.BlockSpec((1, HID), lambda i: (0, 0)),
            pl.BlockSpec((1, HID), lambda i: (0, 0)),
            pl.BlockSpec((HID, LAT), lambda i: (0, 0)),
            pl.BlockSpec((HID, LAT), lambda i: (0, 0)),
            pl.BlockSpec((1, LAT), lambda i: (0, 0)),
            pl.BlockSpec((1, LAT), lambda i: (0, 0)),
        ],
        out_specs=[
            pl.BlockSpec((BR, W2T), lambda i: (i, 0)),
            pl.BlockSpec((BR, LAT), lambda i: (i, 0)),
            pl.BlockSpec((BR, 1), lambda i: (i, 0)),
        ],
        out_shape=[_f32((N_PAD, W2T)), _f32((N_PAD, LAT)), _f32((N_PAD, 1))],
    )(agg1p, res1, g1, b1, w2, r2, as2, ad2)


def _tc_c_body(agg_ref, res_ref, g_ref, b_ref, z_ref):
    agg = agg_ref[0] + agg_ref[1]
    z_ref[...] = _post_layer(agg, res_ref[...], g_ref[...], b_ref[...])


def _tc_c(agg2p, res2, g2, b2):
    return pl.pallas_call(
        _tc_c_body,
        grid=(GRID,),
        in_specs=[
            pl.BlockSpec((NC, BR, W2T), lambda i: (0, i, 0)),
            pl.BlockSpec((BR, LAT), lambda i: (i, 0)),
            pl.BlockSpec((1, LAT), lambda i: (0, 0)),
            pl.BlockSpec((1, LAT), lambda i: (0, 0)),
        ],
        out_specs=pl.BlockSpec((BR, LAT), lambda i: (i, 0)),
        out_shape=_f32((N_PAD, LAT)),
    )(agg2p, res2, g2, b2)


@jax.jit
def kernel(X, edge_index, W1, as1, ad1, R1, g1, b1, W2, as2, ad2, R2, g2, b2):
    s, d = edge_index[0], edge_index[1]
    fill = jnp.full((UE_PAD - UE,), N_PAD - 1, jnp.int32)
    usrc = jnp.concatenate([s, d, fill])
    udst = jnp.concatenate([d, s, fill])

    x_pad = jnp.pad(X, ((0, N_PAD - N_NODES), (0, 0)))

    degp = _deg_kernel(usrc)

    tab1, res1, es1 = _tc_a(x_pad, W1, R1, as1.reshape(1, HID),
                            ad1.reshape(1, HID))
    agg1p = _edge_pass_1(usrc, udst, degp, es1.reshape(N_PAD), tab1)

    tab2, res2, es2 = _tc_b(agg1p, res1, g1.reshape(1, HID),
                            b1.reshape(1, HID), W2, R2,
                            as2.reshape(1, LAT), ad2.reshape(1, LAT))
    agg2p = _edge_pass_2(usrc, udst, degp, es2.reshape(N_PAD), tab2)

    z = _tc_c(agg2p, res2, g2.reshape(1, LAT), b2.reshape(1, LAT))

    scores = _decoder_kernel(usrc, udst, z)
    return scores[:UE]


# speedup vs baseline: 13.3332x; 13.3332x over previous
import functools

import jax
import jax.numpy as jnp
from jax import lax
from jax.experimental import pallas as pl
from jax.experimental.pallas import tpu as pltpu
from jax.experimental.pallas import tpu_sc as plsc

N_NODES = 10000
E = 320000
IN_DIM, HID, LAT = 128, 64, 32
LEAKY_ALPHA = 0.2

NC, NS, L = 2, 16, 16
NW = NC * NS
N_PAD = 10240
UE = 2 * E
C = 128
EPW = 20480
UE_PAD = EPW * NW
NCH = EPW // C
RPT = N_PAD // NS

TW = 128

_mesh = plsc.VectorSubcoreMesh(core_axis_name="c", subcore_axis_name="s",
                               num_cores=NC, num_subcores=NS)
_sc_params = pltpu.CompilerParams(needs_layout_passes=False)

BR = 1024
GRID = N_PAD // BR


def _f32(shape):
    return jax.ShapeDtypeStruct(shape, jnp.float32)


@functools.partial(
    pl.kernel, mesh=_mesh, compiler_params=_sc_params,
    out_type=_f32((NC, N_PAD)),
    scratch_types=[
        pltpu.VMEM_SHARED((N_PAD,), jnp.float32),
        pltpu.VMEM((C,), jnp.int32),
        pltpu.VMEM((C,), jnp.float32),
        pltpu.VMEM((RPT,), jnp.float32),
    ],
)
def _deg_kernel(usrc_hbm, degp_hbm, sh_deg, sidx, ones_v, zbuf):
    c = lax.axis_index("c")
    s = lax.axis_index("s")
    wid = s * NC + c
    sl = pl.ds(s * RPT, RPT)

    @pl.loop(0, RPT // L)
    def _(i):
        zbuf[pl.ds(i * L, L)] = jnp.zeros((L,), jnp.float32)

    @pl.loop(0, C // L)
    def _(i):
        ones_v[pl.ds(i * L, L)] = jnp.ones((L,), jnp.float32)

    pltpu.sync_copy(zbuf, sh_deg.at[sl])
    plsc.subcore_barrier()

    @pl.loop(0, NCH)
    def _(i):
        base = wid * EPW + i * C
        pltpu.sync_copy(usrc_hbm.at[pl.ds(base, C)], sidx)
        pltpu.sync_copy(ones_v, sh_deg.at[sidx], add=True)

    plsc.subcore_barrier()
    pltpu.sync_copy(sh_deg.at[sl], zbuf)
    pltpu.sync_copy(zbuf, degp_hbm.at[c, sl])


def _make_edge_pass(dim):
    col_ed = dim + 1

    @functools.partial(
        pl.kernel, mesh=_mesh, compiler_params=_sc_params,
        out_type=_f32((NC, N_PAD, TW)),
        scratch_types=[
            pltpu.VMEM_SHARED((N_PAD, TW), jnp.float32),
            pltpu.VMEM((N_PAD,), jnp.float32),
            pltpu.VMEM((N_PAD,), jnp.float32),
            pltpu.VMEM((N_PAD,), jnp.float32),
            pltpu.VMEM((C,), jnp.int32),
            pltpu.VMEM((C,), jnp.int32),
            pltpu.VMEM((C, TW), jnp.float32),
            pltpu.SemaphoreType.DMA,
        ],
    )
    def _edge_kernel(usrc_hbm, udst_hbm, degp_hbm, es_hbm, tab_hbm, agg_hbm,
                     sh_agg, es_t, inv_t, tmp_t, sidx, didx, rows, sem):
        c = lax.axis_index("c")
        s = lax.axis_index("s")
        wid = s * NC + c

        pltpu.sync_copy(es_hbm, es_t)
        pltpu.sync_copy(degp_hbm.at[0], inv_t)
        pltpu.sync_copy(degp_hbm.at[1], tmp_t)

        @pl.loop(0, N_PAD // L)
        def _(i):
            dsl = pl.ds(i * L, L)
            dv = inv_t[dsl] + tmp_t[dsl]
            inv_t[dsl] = 1.0 / jnp.maximum(dv, 1.0)

        @pl.loop(0, C)
        def _(e):
            for k in range(TW // L):
                rows[e, pl.ds(k * L, L)] = jnp.zeros((L,), jnp.float32)

        for b in range(RPT // C):
            pltpu.sync_copy(rows, sh_agg.at[pl.ds(s * RPT + b * C, C)])
        plsc.subcore_barrier()

        iota = lax.iota(jnp.int32, L)
        col_v = jnp.full((L,), col_ed, jnp.int32)

        @pl.loop(0, NCH)
        def _(i):
            base = wid * EPW + i * C
            pltpu.sync_copy(usrc_hbm.at[pl.ds(base, C)], sidx)
            pltpu.sync_copy(udst_hbm.at[pl.ds(base, C)], didx)
            pltpu.async_copy(tab_hbm.at[didx], rows, sem).wait()

            @pl.loop(0, C // L)
            def _(j):
                dsl = pl.ds(j * L, L)
                s16 = sidx[dsl]
                r16 = iota + j * L
                es16 = plsc.load_gather(es_t, [s16])
                iv16 = plsc.load_gather(inv_t, [s16])
                ed16 = plsc.load_gather(rows, [r16, col_v])
                t = es16 + ed16
                t = jnp.maximum(t, LEAKY_ALPHA * t)
                exv = jnp.exp(iv16 * t)
                for e in range(L):
                    exs = exv[e]
                    row = j * L + e
                    for k in range(TW // L):
                        ksl = pl.ds(k * L, L)
                        rows[row, ksl] = rows[row, ksl] * exs

            pltpu.sync_copy(rows, sh_agg.at[sidx], add=True)

        plsc.subcore_barrier()
        for b in range(RPT // C):
            bsl = pl.ds(s * RPT + b * C, C)
            pltpu.sync_copy(sh_agg.at[bsl], rows)
            pltpu.sync_copy(rows, agg_hbm.at[c, bsl])

    return _edge_kernel


_edge_pass_1 = _make_edge_pass(HID)
_edge_pass_2 = _make_edge_pass(LAT)


@functools.partial(
    pl.kernel, mesh=_mesh, compiler_params=_sc_params,
    out_type=_f32((UE_PAD,)),
    scratch_types=[
        pltpu.VMEM((C,), jnp.int32),
        pltpu.VMEM((C,), jnp.int32),
        pltpu.VMEM((C, TW), jnp.float32),
        pltpu.VMEM((C, TW), jnp.float32),
        pltpu.VMEM((C,), jnp.float32),
        pltpu.SemaphoreType.DMA,
    ],
)
def _decoder_kernel(usrc_hbm, udst_hbm, z_hbm, out_hbm, sidx, didx,
                    zs, zd, scb, sem):
    c = lax.axis_index("c")
    s = lax.axis_index("s")
    wid = s * NC + c
    iota = lax.iota(jnp.int32, L)

    @pl.loop(0, NCH)
    def _(i):
        base = wid * EPW + i * C
        pltpu.sync_copy(usrc_hbm.at[pl.ds(base, C)], sidx)
        pltpu.sync_copy(udst_hbm.at[pl.ds(base, C)], didx)
        pltpu.async_copy(z_hbm.at[sidx], zs, sem).wait()
        pltpu.async_copy(z_hbm.at[didx], zd, sem).wait()

        @pl.loop(0, C // L)
        def _(j):
            r16 = iota + j * L
            acc = jnp.zeros((L,), jnp.float32)
            for k in range(LAT):
                ck = jnp.full((L,), k, jnp.int32)
                acc = acc + plsc.load_gather(zs, [r16, ck]) * \
                    plsc.load_gather(zd, [r16, ck])
            scb[pl.ds(j * L, L)] = 1.0 / (1.0 + jnp.exp(-acc))

        pltpu.sync_copy(scb, out_hbm.at[pl.ds(base, C)])


def _dot(a, b):
    return lax.dot_general(a, b, (((1,), (0,)), ((), ())),
                           precision=lax.Precision.HIGHEST,
                           preferred_element_type=jnp.float32)


def _tc_a_body(x_ref, w_ref, r_ref, as_ref, ad_ref, tab_ref, res_ref, es_ref):
    x = x_ref[...]
    h = _dot(x, w_ref[...])
    res_ref[...] = _dot(x, r_ref[...])
    es_ref[...] = jnp.sum(h * as_ref[...], axis=1, keepdims=True)
    ed = jnp.sum(h * ad_ref[...], axis=1, keepdims=True)
    ones = jnp.ones((BR, 1), jnp.float32)
    pad = jnp.zeros((BR, TW - HID - 2), jnp.float32)
    tab_ref[...] = jnp.concatenate([h, ones, ed, pad], axis=1)


def _tc_a(x_pad, w1, r1, as1, ad1):
    return pl.pallas_call(
        _tc_a_body,
        grid=(GRID,),
        in_specs=[
            pl.BlockSpec((BR, IN_DIM), lambda i: (i, 0)),
            pl.BlockSpec((IN_DIM, HID), lambda i: (0, 0)),
            pl.BlockSpec((IN_DIM, HID), lambda i: (0, 0)),
            pl.BlockSpec((1, HID), lambda i: (0, 0)),
            pl.BlockSpec((1, HID), lambda i: (0, 0)),
        ],
        out_specs=[
            pl.BlockSpec((BR, TW), lambda i: (i, 0)),
            pl.BlockSpec((BR, HID), lambda i: (i, 0)),
            pl.BlockSpec((BR, 1), lambda i: (i, 0)),
        ],
        out_shape=[_f32((N_PAD, TW)), _f32((N_PAD, HID)), _f32((N_PAD, 1))],
    )(x_pad, w1, r1, as1, ad1)


def _post_layer(agg, res, g, b):
    dim = res.shape[1]
    num = agg[:, :dim]
    den = agg[:, dim:dim + 1]
    h = num / (den + 1e-16) + res
    mu = jnp.mean(h, axis=1, keepdims=True)
    d = h - mu
    var = jnp.mean(d * d, axis=1, keepdims=True)
    hn = d * lax.rsqrt(var + 1e-5) * g + b
    return jnp.where(hn > 0, hn, jnp.exp(hn) - 1.0)


def _tc_b_body(agg_ref, res_ref, g_ref, b_ref, w_ref, r_ref, as_ref, ad_ref,
               tab_ref, res2_ref, es_ref):
    agg = agg_ref[0] + agg_ref[1]
    x2 = _post_layer(agg, res_ref[...], g_ref[...], b_ref[...])
    h2 = _dot(x2, w_ref[...])
    res2_ref[...] = _dot(x2, r_ref[...])
    es_ref[...] = jnp.sum(h2 * as_ref[...], axis=1, keepdims=True)
    ed = jnp.sum(h2 * ad_ref[...], axis=1, keepdims=True)
    ones = jnp.ones((BR, 1), jnp.float32)
    pad = jnp.zeros((BR, TW - LAT - 2), jnp.float32)
    tab_ref[...] = jnp.concatenate([h2, ones, ed, pad], axis=1)


def _tc_b(agg1p, res1, g1, b1, w2, r2, as2, ad2):
    return pl.pallas_call(
        _tc_b_body,
        grid=(GRID,),
        in_specs=[
            pl.BlockSpec((NC, BR, TW), lambda i: (0, i, 0)),
            pl.BlockSpec((BR, HID), lambda i: (i, 0)),
            pl.BlockSpec((1, HID), lambda i: (0, 0)),
            pl.BlockSpec((1, HID), lambda i: (0, 0)),
            pl.BlockSpec((HID, LAT), lambda i: (0, 0)),
            pl.BlockSpec((HID, LAT), lambda i: (0, 0)),
            pl.BlockSpec((1, LAT), lambda i: (0, 0)),
            pl.BlockSpec((1, LAT), lambda i: (0, 0)),
        ],
        out_specs=[
            pl.BlockSpec((BR, TW), lambda i: (i, 0)),
            pl.BlockSpec((BR, LAT), lambda i: (i, 0)),
            pl.BlockSpec((BR, 1), lambda i: (i, 0)),
        ],
        out_shape=[_f32((N_PAD, TW)), _f32((N_PAD, LAT)), _f32((N_PAD, 1))],
    )(agg1p, res1, g1, b1, w2, r2, as2, ad2)


def _tc_c_body(agg_ref, res_ref, g_ref, b_ref, z_ref):
    agg = agg_ref[0] + agg_ref[1]
    zv = _post_layer(agg, res_ref[...], g_ref[...], b_ref[...])
    z_ref[...] = jnp.concatenate(
        [zv, jnp.zeros((BR, TW - LAT), jnp.float32)], axis=1)


def _tc_c(agg2p, res2, g2, b2):
    return pl.pallas_call(
        _tc_c_body,
        grid=(GRID,),
        in_specs=[
            pl.BlockSpec((NC, BR, TW), lambda i: (0, i, 0)),
            pl.BlockSpec((BR, LAT), lambda i: (i, 0)),
            pl.BlockSpec((1, LAT), lambda i: (0, 0)),
            pl.BlockSpec((1, LAT), lambda i: (0, 0)),
        ],
        out_specs=pl.BlockSpec((BR, TW), lambda i: (i, 0)),
        out_shape=_f32((N_PAD, TW)),
    )(agg2p, res2, g2, b2)


@jax.jit
def kernel(X, edge_index, W1, as1, ad1, R1, g1, b1, W2, as2, ad2, R2, g2, b2):
    s, d = edge_index[0], edge_index[1]
    fill = jnp.full((UE_PAD - UE,), N_PAD - 1, jnp.int32)
    usrc = jnp.concatenate([s, d, fill])
    udst = jnp.concatenate([d, s, fill])

    x_pad = jnp.pad(X, ((0, N_PAD - N_NODES), (0, 0)))

    degp = _deg_kernel(usrc)

    tab1, res1, es1 = _tc_a(x_pad, W1, R1, as1.reshape(1, HID),
                            ad1.reshape(1, HID))
    agg1p = _edge_pass_1(usrc, udst, degp, es1.reshape(N_PAD), tab1)

    tab2, res2, es2 = _tc_b(agg1p, res1, g1.reshape(1, HID),
                            b1.reshape(1, HID), W2, R2,
                            as2.reshape(1, LAT), ad2.reshape(1, LAT))
    agg2p = _edge_pass_2(usrc, udst, degp, es2.reshape(N_PAD), tab2)

    z = _tc_c(agg2p, res2, g2.reshape(1, LAT), b2.reshape(1, LAT))

    scores = _decoder_kernel(usrc, udst, z)
    return scores[:UE]


# speedup vs baseline: 13.5661x; 1.0175x over previous
import functools

import jax
import jax.numpy as jnp
from jax import lax
from jax.experimental import pallas as pl
from jax.experimental.pallas import tpu as pltpu
from jax.experimental.pallas import tpu_sc as plsc

N_NODES = 10000
E = 320000
IN_DIM, HID, LAT = 128, 64, 32
LEAKY_ALPHA = 0.2

NC, NS, L = 2, 16, 16
NW = NC * NS
N_PAD = 10240
UE = 2 * E
C = 128
EPW = 20480
UE_PAD = EPW * NW
NCH = EPW // C
RPT = N_PAD // NS

TW = 128

_mesh = plsc.VectorSubcoreMesh(core_axis_name="c", subcore_axis_name="s",
                               num_cores=NC, num_subcores=NS)
_sc_params = pltpu.CompilerParams(needs_layout_passes=False)

BR = 1024
GRID = N_PAD // BR


def _f32(shape):
    return jax.ShapeDtypeStruct(shape, jnp.float32)


@functools.partial(
    pl.kernel, mesh=_mesh, compiler_params=_sc_params,
    out_type=_f32((NC, N_PAD)),
    scratch_types=[
        pltpu.VMEM_SHARED((N_PAD,), jnp.float32),
        pltpu.VMEM((C,), jnp.int32),
        pltpu.VMEM((C,), jnp.float32),
        pltpu.VMEM((RPT,), jnp.float32),
    ],
)
def _deg_kernel(usrc_hbm, degp_hbm, sh_deg, sidx, ones_v, zbuf):
    c = lax.axis_index("c")
    s = lax.axis_index("s")
    wid = s * NC + c
    sl = pl.ds(s * RPT, RPT)

    @pl.loop(0, RPT // L)
    def _(i):
        zbuf[pl.ds(i * L, L)] = jnp.zeros((L,), jnp.float32)

    @pl.loop(0, C // L)
    def _(i):
        ones_v[pl.ds(i * L, L)] = jnp.ones((L,), jnp.float32)

    pltpu.sync_copy(zbuf, sh_deg.at[sl])
    plsc.subcore_barrier()

    @pl.loop(0, NCH)
    def _(i):
        base = wid * EPW + i * C
        pltpu.sync_copy(usrc_hbm.at[pl.ds(base, C)], sidx)
        pltpu.sync_copy(ones_v, sh_deg.at[sidx], add=True)

    plsc.subcore_barrier()
    pltpu.sync_copy(sh_deg.at[sl], zbuf)
    pltpu.sync_copy(zbuf, degp_hbm.at[c, sl])


def _make_edge_pass(dim):
    col_ed = dim + 1
    ks = (dim + 2 + L - 1) // L

    @functools.partial(
        pl.kernel, mesh=_mesh, compiler_params=_sc_params,
        out_type=_f32((NC, N_PAD, TW)),
        scratch_types=[
            pltpu.VMEM_SHARED((N_PAD, TW), jnp.float32),
            pltpu.VMEM((N_PAD,), jnp.float32),
            pltpu.VMEM((N_PAD,), jnp.float32),
            pltpu.VMEM((N_PAD,), jnp.float32),
            pltpu.VMEM((C,), jnp.int32),
            pltpu.VMEM((C,), jnp.int32),
            pltpu.VMEM((C, TW), jnp.float32),
            pltpu.SemaphoreType.DMA,
        ],
    )
    def _edge_kernel(usrc_hbm, udst_hbm, degp_hbm, es_hbm, tab_hbm, agg_hbm,
                     sh_agg, es_t, inv_t, tmp_t, sidx, didx, rows, sem):
        c = lax.axis_index("c")
        s = lax.axis_index("s")
        wid = s * NC + c

        pltpu.sync_copy(es_hbm, es_t)
        pltpu.sync_copy(degp_hbm.at[0], inv_t)
        pltpu.sync_copy(degp_hbm.at[1], tmp_t)

        @pl.loop(0, N_PAD // L)
        def _(i):
            dsl = pl.ds(i * L, L)
            dv = inv_t[dsl] + tmp_t[dsl]
            inv_t[dsl] = 1.0 / jnp.maximum(dv, 1.0)

        @pl.loop(0, C)
        def _(e):
            for k in range(TW // L):
                rows[e, pl.ds(k * L, L)] = jnp.zeros((L,), jnp.float32)

        for b in range(RPT // C):
            pltpu.sync_copy(rows, sh_agg.at[pl.ds(s * RPT + b * C, C)])
        plsc.subcore_barrier()

        iota = lax.iota(jnp.int32, L)
        col_v = jnp.full((L,), col_ed, jnp.int32)

        @pl.loop(0, NCH)
        def _(i):
            base = wid * EPW + i * C
            pltpu.sync_copy(usrc_hbm.at[pl.ds(base, C)], sidx)
            pltpu.sync_copy(udst_hbm.at[pl.ds(base, C)], didx)
            pltpu.async_copy(tab_hbm.at[didx], rows, sem).wait()

            @pl.loop(0, C // L)
            def _(j):
                dsl = pl.ds(j * L, L)
                s16 = sidx[dsl]
                r16 = iota + j * L
                es16 = plsc.load_gather(es_t, [s16])
                iv16 = plsc.load_gather(inv_t, [s16])
                ed16 = plsc.load_gather(rows, [r16, col_v])
                t = es16 + ed16
                t = jnp.maximum(t, LEAKY_ALPHA * t)
                exv = jnp.exp(iv16 * t)
                for e in range(L):
                    exs = exv[e]
                    row = j * L + e
                    for k in range(ks):
                        ksl = pl.ds(k * L, L)
                        rows[row, ksl] = rows[row, ksl] * exs

            pltpu.sync_copy(rows, sh_agg.at[sidx], add=True)

        plsc.subcore_barrier()
        for b in range(RPT // C):
            bsl = pl.ds(s * RPT + b * C, C)
            pltpu.sync_copy(sh_agg.at[bsl], rows)
            pltpu.sync_copy(rows, agg_hbm.at[c, bsl])

    return _edge_kernel


_edge_pass_1 = _make_edge_pass(HID)
_edge_pass_2 = _make_edge_pass(LAT)


@functools.partial(
    pl.kernel, mesh=_mesh, compiler_params=_sc_params,
    out_type=_f32((UE_PAD,)),
    scratch_types=[
        pltpu.VMEM((C,), jnp.int32),
        pltpu.VMEM((C,), jnp.int32),
        pltpu.VMEM((C, TW), jnp.float32),
        pltpu.VMEM((C, TW), jnp.float32),
        pltpu.VMEM((C,), jnp.float32),
        pltpu.SemaphoreType.DMA,
    ],
)
def _decoder_kernel(usrc_hbm, udst_hbm, z_hbm, out_hbm, sidx, didx,
                    zs, zd, scb, sem):
    c = lax.axis_index("c")
    s = lax.axis_index("s")
    wid = s * NC + c
    iota = lax.iota(jnp.int32, L)

    @pl.loop(0, NCH)
    def _(i):
        base = wid * EPW + i * C
        pltpu.sync_copy(usrc_hbm.at[pl.ds(base, C)], sidx)
        pltpu.sync_copy(udst_hbm.at[pl.ds(base, C)], didx)
        pltpu.async_copy(z_hbm.at[sidx], zs, sem).wait()
        pltpu.async_copy(z_hbm.at[didx], zd, sem).wait()

        @pl.loop(0, C // L)
        def _(j):
            r16 = iota + j * L
            acc = jnp.zeros((L,), jnp.float32)
            for k in range(LAT):
                ck = jnp.full((L,), k, jnp.int32)
                acc = acc + plsc.load_gather(zs, [r16, ck]) * \
                    plsc.load_gather(zd, [r16, ck])
            scb[pl.ds(j * L, L)] = 1.0 / (1.0 + jnp.exp(-acc))

        pltpu.sync_copy(scb, out_hbm.at[pl.ds(base, C)])


def _dot(a, b):
    return lax.dot_general(a, b, (((1,), (0,)), ((), ())),
                           precision=lax.Precision.HIGHEST,
                           preferred_element_type=jnp.float32)


def _tc_a_body(x_ref, w_ref, r_ref, as_ref, ad_ref, tab_ref, res_ref, es_ref):
    x = x_ref[...]
    h = _dot(x, w_ref[...])
    res_ref[...] = _dot(x, r_ref[...])
    es_ref[...] = jnp.sum(h * as_ref[...], axis=1, keepdims=True)
    ed = jnp.sum(h * ad_ref[...], axis=1, keepdims=True)
    ones = jnp.ones((BR, 1), jnp.float32)
    pad = jnp.zeros((BR, TW - HID - 2), jnp.float32)
    tab_ref[...] = jnp.concatenate([h, ones, ed, pad], axis=1)


def _tc_a(x_pad, w1, r1, as1, ad1):
    return pl.pallas_call(
        _tc_a_body,
        grid=(GRID,),
        in_specs=[
            pl.BlockSpec((BR, IN_DIM), lambda i: (i, 0)),
            pl.BlockSpec((IN_DIM, HID), lambda i: (0, 0)),
            pl.BlockSpec((IN_DIM, HID), lambda i: (0, 0)),
            pl.BlockSpec((1, HID), lambda i: (0, 0)),
            pl.BlockSpec((1, HID), lambda i: (0, 0)),
        ],
        out_specs=[
            pl.BlockSpec((BR, TW), lambda i: (i, 0)),
            pl.BlockSpec((BR, HID), lambda i: (i, 0)),
            pl.BlockSpec((BR, 1), lambda i: (i, 0)),
        ],
        out_shape=[_f32((N_PAD, TW)), _f32((N_PAD, HID)), _f32((N_PAD, 1))],
    )(x_pad, w1, r1, as1, ad1)


def _post_layer(agg, res, g, b):
    dim = res.shape[1]
    num = agg[:, :dim]
    den = agg[:, dim:dim + 1]
    h = num / (den + 1e-16) + res
    mu = jnp.mean(h, axis=1, keepdims=True)
    d = h - mu
    var = jnp.mean(d * d, axis=1, keepdims=True)
    hn = d * lax.rsqrt(var + 1e-5) * g + b
    return jnp.where(hn > 0, hn, jnp.exp(hn) - 1.0)


def _tc_b_body(agg_ref, res_ref, g_ref, b_ref, w_ref, r_ref, as_ref, ad_ref,
               tab_ref, res2_ref, es_ref):
    agg = agg_ref[0] + agg_ref[1]
    x2 = _post_layer(agg, res_ref[...], g_ref[...], b_ref[...])
    h2 = _dot(x2, w_ref[...])
    res2_ref[...] = _dot(x2, r_ref[...])
    es_ref[...] = jnp.sum(h2 * as_ref[...], axis=1, keepdims=True)
    ed = jnp.sum(h2 * ad_ref[...], axis=1, keepdims=True)
    ones = jnp.ones((BR, 1), jnp.float32)
    pad = jnp.zeros((BR, TW - LAT - 2), jnp.float32)
    tab_ref[...] = jnp.concatenate([h2, ones, ed, pad], axis=1)


def _tc_b(agg1p, res1, g1, b1, w2, r2, as2, ad2):
    return pl.pallas_call(
        _tc_b_body,
        grid=(GRID,),
        in_specs=[
            pl.BlockSpec((NC, BR, TW), lambda i: (0, i, 0)),
            pl.BlockSpec((BR, HID), lambda i: (i, 0)),
            pl.BlockSpec((1, HID), lambda i: (0, 0)),
            pl.BlockSpec((1, HID), lambda i: (0, 0)),
            pl.BlockSpec((HID, LAT), lambda i: (0, 0)),
            pl.BlockSpec((HID, LAT), lambda i: (0, 0)),
            pl.BlockSpec((1, LAT), lambda i: (0, 0)),
            pl.BlockSpec((1, LAT), lambda i: (0, 0)),
        ],
        out_specs=[
            pl.BlockSpec((BR, TW), lambda i: (i, 0)),
            pl.BlockSpec((BR, LAT), lambda i: (i, 0)),
            pl.BlockSpec((BR, 1), lambda i: (i, 0)),
        ],
        out_shape=[_f32((N_PAD, TW)), _f32((N_PAD, LAT)), _f32((N_PAD, 1))],
    )(agg1p, res1, g1, b1, w2, r2, as2, ad2)


def _tc_c_body(agg_ref, res_ref, g_ref, b_ref, z_ref):
    agg = agg_ref[0] + agg_ref[1]
    zv = _post_layer(agg, res_ref[...], g_ref[...], b_ref[...])
    z_ref[...] = jnp.concatenate(
        [zv, jnp.zeros((BR, TW - LAT), jnp.float32)], axis=1)


def _tc_c(agg2p, res2, g2, b2):
    return pl.pallas_call(
        _tc_c_body,
        grid=(GRID,),
        in_specs=[
            pl.BlockSpec((NC, BR, TW), lambda i: (0, i, 0)),
            pl.BlockSpec((BR, LAT), lambda i: (i, 0)),
            pl.BlockSpec((1, LAT), lambda i: (0, 0)),
            pl.BlockSpec((1, LAT), lambda i: (0, 0)),
        ],
        out_specs=pl.BlockSpec((BR, TW), lambda i: (i, 0)),
        out_shape=_f32((N_PAD, TW)),
    )(agg2p, res2, g2, b2)


@jax.jit
def kernel(X, edge_index, W1, as1, ad1, R1, g1, b1, W2, as2, ad2, R2, g2, b2):
    s, d = edge_index[0], edge_index[1]
    fill = jnp.full((UE_PAD - UE,), N_PAD - 1, jnp.int32)
    usrc = jnp.concatenate([s, d, fill])
    udst = jnp.concatenate([d, s, fill])

    x_pad = jnp.pad(X, ((0, N_PAD - N_NODES), (0, 0)))

    degp = _deg_kernel(usrc)

    tab1, res1, es1 = _tc_a(x_pad, W1, R1, as1.reshape(1, HID),
                            ad1.reshape(1, HID))
    agg1p = _edge_pass_1(usrc, udst, degp, es1.reshape(N_PAD), tab1)

    tab2, res2, es2 = _tc_b(agg1p, res1, g1.reshape(1, HID),
                            b1.reshape(1, HID), W2, R2,
                            as2.reshape(1, LAT), ad2.reshape(1, LAT))
    agg2p = _edge_pass_2(usrc, udst, degp, es2.reshape(N_PAD), tab2)

    z = _tc_c(agg2p, res2, g2.reshape(1, LAT), b2.reshape(1, LAT))

    scores = _decoder_kernel(usrc, udst, z)
    return scores[:UE]
